# baseline (speedup 1.0000x reference)
import functools

import jax
import jax.numpy as jnp
from jax import lax
from jax.experimental import pallas as pl
from jax.experimental.pallas import tpu as pltpu
from jax.experimental.pallas import tpu_sc as plsc

_V = 100
_D = 80
_B = 4096 * 200
_NC, _NS = 2, 16
_NW = _NC * _NS
_RPW = _B // _NW
_CHUNK = 512
_CPW = _RPW // _CHUNK
_NBUF = 2
_G = _CHUNK // 16

_mesh = plsc.VectorSubcoreMesh(core_axis_name="c", subcore_axis_name="s")


@functools.partial(
    pl.kernel,
    mesh=_mesh,
    out_type=jax.ShapeDtypeStruct((_B * _D,), jnp.float32),
    compiler_params=pltpu.CompilerParams(
        use_tc_tiling_on_sc=False, needs_layout_passes=False),
    scratch_types=[
        pltpu.VMEM((_RPW,), jnp.int32),
        pltpu.VMEM((_V * _D,), jnp.float32),
        pltpu.VMEM((_NBUF, _CHUNK * _D), jnp.float32),
        pltpu.SemaphoreType.DMA((_NBUF,)),
    ],
)
def _emb_lookup(idx_hbm, table_hbm, out_hbm, idx_v, table_v, rows_v, osem):
    wid = lax.axis_index("s") * _NC + lax.axis_index("c")
    rbase = wid * _RPW
    pltpu.sync_copy(table_hbm, table_v)
    pltpu.sync_copy(idx_hbm.at[pl.ds(rbase, _RPW)], idx_v)

    def outcp(c, b):
        return pltpu.make_async_copy(
            rows_v.at[b],
            out_hbm.at[pl.ds((rbase + c * _CHUNK) * _D, _CHUNK * _D)],
            osem.at[b])

    def compute(c, b):
        def grp(k, carry):
            idxv = idx_v[pl.ds(c * _CHUNK + k * 16, 16)]
            addr = idxv * _D
            dst = lax.iota(jnp.int32, 16) * _D + k * (16 * _D)
            for col in range(_D):
                x = plsc.load_gather(table_v, [addr + col])
                plsc.store_scatter(rows_v.at[b], [dst + col], x)
            return carry

        lax.fori_loop(0, _G, grp, 0)

    def outer(i, carry):
        for b in range(_NBUF):
            c = i * _NBUF + b

            @pl.when(c >= _NBUF)
            def _():
                outcp(c - _NBUF, b).wait()

            compute(c, b)
            outcp(c, b).start()
        return carry

    lax.fori_loop(0, _CPW // _NBUF, outer, 0)

    for b in range(_NBUF):
        outcp(_CPW - _NBUF + b, b).wait()


def kernel(phone, table):
    out = _emb_lookup(phone.reshape(-1), table.reshape(-1))
    return out.reshape(phone.shape + (table.shape[1],))


# speedup vs baseline: 1.6094x; 1.6094x over previous
import functools

import jax
import jax.numpy as jnp
from jax import lax
from jax.experimental import pallas as pl
from jax.experimental.pallas import tpu as pltpu
from jax.experimental.pallas import tpu_sc as plsc

_V = 100
_D = 80
_B = 4096 * 200
_NC, _NS = 2, 16
_NW = _NC * _NS
_RPW = _B // _NW
_CHUNK = 512
_CPW = _RPW // _CHUNK
_NBUF = 2
_U = 8

_mesh = plsc.VectorSubcoreMesh(core_axis_name="c", subcore_axis_name="s")


@functools.partial(
    pl.kernel,
    mesh=_mesh,
    out_type=jax.ShapeDtypeStruct((_B * _D,), jnp.float32),
    compiler_params=pltpu.CompilerParams(
        use_tc_tiling_on_sc=False, needs_layout_passes=False),
    scratch_types=[
        pltpu.VMEM((_RPW,), jnp.int32),
        pltpu.VMEM((_V * _D,), jnp.float32),
        pltpu.VMEM((_NBUF, _CHUNK * _D), jnp.float32),
        pltpu.SemaphoreType.DMA((_NBUF,)),
    ],
)
def _emb_lookup(idx_hbm, table_hbm, out_hbm, idx_v, table_v, rows_v, osem):
    wid = lax.axis_index("s") * _NC + lax.axis_index("c")
    rbase = wid * _RPW
    pltpu.sync_copy(table_hbm, table_v)
    pltpu.sync_copy(idx_hbm.at[pl.ds(rbase, _RPW)], idx_v)

    def outcp(c, b):
        return pltpu.make_async_copy(
            rows_v.at[b],
            out_hbm.at[pl.ds((rbase + c * _CHUNK) * _D, _CHUNK * _D)],
            osem.at[b])

    def compute(c, b):
        def rows(j, carry):
            idxv = idx_v[pl.ds(c * _CHUNK + j * 16, 16)]
            for u in range(16):
                src = idxv[u] * _D
                dst = (j * 16 + u) * _D
                for col in range(_D // 16):
                    rows_v.at[b][pl.ds(dst + col * 16, 16)] = (
                        table_v[pl.ds(src + col * 16, 16)])
            return carry

        lax.fori_loop(0, _CHUNK // 16, rows, 0)

    def outer(i, carry):
        for b in range(_NBUF):
            c = i * _NBUF + b

            @pl.when(c >= _NBUF)
            def _():
                outcp(c - _NBUF, b).wait()

            compute(c, b)
            outcp(c, b).start()
        return carry

    lax.fori_loop(0, _CPW // _NBUF, outer, 0)

    for b in range(_NBUF):
        outcp(_CPW - _NBUF + b, b).wait()


def kernel(phone, table):
    out = _emb_lookup(phone.reshape(-1), table.reshape(-1))
    return out.reshape(phone.shape + (table.shape[1],))


# speedup vs baseline: 1.7251x; 1.0718x over previous
import functools

import jax
import jax.numpy as jnp
from jax import lax
from jax.experimental import pallas as pl
from jax.experimental.pallas import tpu as pltpu
from jax.experimental.pallas import tpu_sc as plsc

_V = 100
_D = 80
_P = 81
_B = 4096 * 200
_NC, _NS = 2, 16
_NW = _NC * _NS
_RPW = _B // _NW
_CHUNK = 512
_CPW = _RPW // _CHUNK
_NBUF = 2

_mesh = plsc.VectorSubcoreMesh(core_axis_name="c", subcore_axis_name="s")


@functools.partial(
    pl.kernel,
    mesh=_mesh,
    out_type=jax.ShapeDtypeStruct((_B, _D), jnp.float32),
    compiler_params=pltpu.CompilerParams(
        use_tc_tiling_on_sc=False, needs_layout_passes=False),
    scratch_types=[
        pltpu.VMEM((_RPW,), jnp.int32),
        pltpu.VMEM((_V * _P,), jnp.float32),
        pltpu.VMEM((_NBUF, _CHUNK, _P), jnp.float32),
        pltpu.SemaphoreType.DMA((_NBUF,)),
    ],
)
def _emb_lookup(idx_hbm, table_hbm, out_hbm, idx_v, table_v, rows_v, osem):
    wid = lax.axis_index("s") * _NC + lax.axis_index("c")
    rbase = wid * _RPW
    pltpu.sync_copy(table_hbm, table_v)
    pltpu.sync_copy(idx_hbm.at[pl.ds(rbase, _RPW)], idx_v)

    def outcp(c, b):
        return pltpu.make_async_copy(
            rows_v.at[b, :, pl.ds(0, _D)],
            out_hbm.at[pl.ds(rbase + c * _CHUNK, _CHUNK), :],
            osem.at[b])

    def compute(c, b):
        @plsc.parallel_loop(0, _CHUNK // 16, 1)
        def _(j):
            idxv = idx_v[pl.ds(c * _CHUNK + j * 16, 16)]
            addr = idxv * _P
            rowv = lax.iota(jnp.int32, 16) + j * 16
            colv = lax.iota(jnp.int32, 16) * 0
            for col in range(_D):
                x = plsc.load_gather(table_v, [addr + col])
                plsc.store_scatter(rows_v.at[b], [rowv, colv + col], x)

    def outer(i, carry):
        for b in range(_NBUF):
            c = i * _NBUF + b

            @pl.when(c >= _NBUF)
            def _():
                outcp(c - _NBUF, b).wait()

            compute(c, b)
            outcp(c, b).start()
        return carry

    lax.fori_loop(0, _CPW // _NBUF, outer, 0)

    for b in range(_NBUF):
        outcp(_CPW - _NBUF + b, b).wait()


def kernel(phone, table):
    tpad = jnp.pad(table, ((0, 0), (0, _P - _D)))
    out = _emb_lookup(phone.reshape(-1), tpad.reshape(-1))
    return out.reshape(phone.shape + (table.shape[1],))


# speedup vs baseline: 2.3411x; 1.3571x over previous
import functools

import jax
import jax.numpy as jnp
from jax import lax
from jax.experimental import pallas as pl
from jax.experimental.pallas import tpu as pltpu
from jax.experimental.pallas import tpu_sc as plsc

_V = 100
_D = 80
_P = 81
_B = 4096 * 200
_NC, _NS = 2, 16
_NW = _NC * _NS
_RPW = _B // _NW
_CHUNK = 512
_CPW = _RPW // _CHUNK
_NBUF = 2

_mesh = plsc.VectorSubcoreMesh(core_axis_name="c", subcore_axis_name="s")


@functools.partial(
    pl.kernel,
    mesh=_mesh,
    out_type=jax.ShapeDtypeStruct((_B * _D,), jnp.float32),
    compiler_params=pltpu.CompilerParams(
        use_tc_tiling_on_sc=False, needs_layout_passes=False),
    scratch_types=[
        pltpu.VMEM((_RPW,), jnp.int32),
        pltpu.VMEM((_V * _P,), jnp.float32),
        pltpu.VMEM((_NBUF, _CHUNK * _D), jnp.float32),
        pltpu.SemaphoreType.DMA((_NBUF,)),
    ],
)
def _emb_lookup(idx_hbm, table_hbm, out_hbm, idx_v, table_v, rows_v, osem):
    wid = lax.axis_index("s") * _NC + lax.axis_index("c")
    rbase = wid * _RPW
    pltpu.sync_copy(table_hbm, table_v)
    pltpu.sync_copy(idx_hbm.at[pl.ds(rbase, _RPW)], idx_v)

    def outcp(c, b):
        return pltpu.make_async_copy(
            rows_v.at[b],
            out_hbm.at[pl.ds((rbase + c * _CHUNK) * _D, _CHUNK * _D)],
            osem.at[b])

    def compute(c, b):
        @plsc.parallel_loop(0, _CHUNK // 16, 1)
        def _(j):
            idxv = idx_v[pl.ds(c * _CHUNK + j * 16, 16)] * _P
            for u in range(16):
                src = idxv[u]
                dst = (j * 16 + u) * _D
                for col in range(_D // 16):
                    rows_v.at[b][pl.ds(dst + col * 16, 16)] = (
                        table_v[pl.ds(src + col * 16, 16)])

    def outer(i, carry):
        for b in range(_NBUF):
            c = i * _NBUF + b

            @pl.when(c >= _NBUF)
            def _():
                outcp(c - _NBUF, b).wait()

            compute(c, b)
            outcp(c, b).start()
        return carry

    lax.fori_loop(0, _CPW // _NBUF, outer, 0)

    for b in range(_NBUF):
        outcp(_CPW - _NBUF + b, b).wait()


def kernel(phone, table):
    tpad = jnp.pad(table, ((0, 0), (0, _P - _D)))
    out = _emb_lookup(phone.reshape(-1), tpad.reshape(-1))
    return out.reshape(phone.shape + (table.shape[1],))
